# baseline (speedup 1.0000x reference)
import functools

import jax
import jax.numpy as jnp
from jax import lax
from jax.experimental import pallas as pl
from jax.experimental.pallas import tpu as pltpu
from jax.experimental.pallas import tpu_sc as plsc

N = 10000
E = 160000
IN_DIM = 256
HID = 128
H = 8
C = 40
SLOPE = 0.2

NC = 2
NS = 16
NW = NC * NS
EPW = E // NW
CH = 128
NCHUNK = -(-EPW // CH)
EPW_PAD = NCHUNK * CH
NPAD = 10112
ROWS_PER_TILE = NPAD // NS

MB = 1000
GRID = N // MB


def _tc1_body(x_ref, w_ref, a_ref, ft_ref, elr_ref):
    feat = jnp.dot(x_ref[...], w_ref[...], preferred_element_type=jnp.float32)
    ee = jnp.dot(feat, a_ref[...], preferred_element_type=jnp.float32)
    elr_ref[...] = jnp.concatenate(
        [ee, jnp.zeros((MB, HID - 32), jnp.float32)], axis=1)
    for h in range(H):
        ft_ref[h] = feat[:, h * HID:(h + 1) * HID]


def _tc1(x, W, A):
    return pl.pallas_call(
        _tc1_body,
        grid=(GRID,),
        in_specs=[
            pl.BlockSpec((MB, x.shape[1]), lambda m: (m, 0)),
            pl.BlockSpec((W.shape[0], H * HID), lambda m: (0, 0)),
            pl.BlockSpec((H * HID, 32), lambda m: (0, 0)),
        ],
        out_specs=[
            pl.BlockSpec((H, MB, HID), lambda m: (0, m, 0)),
            pl.BlockSpec((MB, HID), lambda m: (m, 0)),
        ],
        out_shape=[
            jax.ShapeDtypeStruct((H, N, HID), jnp.float32),
            jax.ShapeDtypeStruct((N, HID), jnp.float32),
        ],
    )(x, W, A)


def _combine(msg_p, den_p, b, res):
    den = den_p[0] + den_p[1]
    parts = []
    for h in range(H):
        d = den[:, h:h + 1]
        d = jnp.where(d == 0.0, 1.0, d)
        xh = (msg_p[0, h] + msg_p[1, h]) / d + b[h:h + 1, :]
        if res is not None:
            xh = xh + res[:, h * HID:(h + 1) * HID]
        parts.append(jnp.where(xh > 0, xh, jnp.exp(jnp.minimum(xh, 0.0)) - 1.0))
    return jnp.concatenate(parts, axis=1)


def _tc2_body(mp_ref, dp_ref, b_ref, w_ref, a_ref, h1_ref, ft_ref, elr_ref):
    h1 = _combine(mp_ref[...], dp_ref[...], b_ref[...], None)
    h1_ref[...] = h1
    feat = jnp.dot(h1, w_ref[...], preferred_element_type=jnp.float32)
    ee = jnp.dot(feat, a_ref[...], preferred_element_type=jnp.float32)
    elr_ref[...] = jnp.concatenate(
        [ee, jnp.zeros((MB, HID - 32), jnp.float32)], axis=1)
    for h in range(H):
        ft_ref[h] = feat[:, h * HID:(h + 1) * HID]


def _tc2(msg_p, den_p, b, W, A):
    return pl.pallas_call(
        _tc2_body,
        grid=(GRID,),
        in_specs=[
            pl.BlockSpec((2, H, MB, HID), lambda m: (0, 0, m, 0)),
            pl.BlockSpec((2, MB, HID), lambda m: (0, m, 0)),
            pl.BlockSpec((H, HID), lambda m: (0, 0)),
            pl.BlockSpec((H * HID, H * HID), lambda m: (0, 0)),
            pl.BlockSpec((H * HID, 32), lambda m: (0, 0)),
        ],
        out_specs=[
            pl.BlockSpec((MB, H * HID), lambda m: (m, 0)),
            pl.BlockSpec((H, MB, HID), lambda m: (0, m, 0)),
            pl.BlockSpec((MB, HID), lambda m: (m, 0)),
        ],
        out_shape=[
            jax.ShapeDtypeStruct((N, H * HID), jnp.float32),
            jax.ShapeDtypeStruct((H, N, HID), jnp.float32),
            jax.ShapeDtypeStruct((N, HID), jnp.float32),
        ],
    )(msg_p, den_p, b, W, A)


def _tc3_body(mp_ref, dp_ref, res_ref, b_ref, wc_ref, bc_ref, h_ref, lg_ref):
    hh = _combine(mp_ref[...], dp_ref[...], b_ref[...], res_ref[...])
    h_ref[...] = hh
    lg_ref[...] = jnp.dot(hh, wc_ref[...], preferred_element_type=jnp.float32) + bc_ref[...]


def _tc3(msg_p, den_p, res, b, Wc, bc):
    return pl.pallas_call(
        _tc3_body,
        grid=(GRID,),
        in_specs=[
            pl.BlockSpec((2, H, MB, HID), lambda m: (0, 0, m, 0)),
            pl.BlockSpec((2, MB, HID), lambda m: (0, m, 0)),
            pl.BlockSpec((MB, H * HID), lambda m: (m, 0)),
            pl.BlockSpec((H, HID), lambda m: (0, 0)),
            pl.BlockSpec((H * HID, C), lambda m: (0, 0)),
            pl.BlockSpec((1, C), lambda m: (0, 0)),
        ],
        out_specs=[
            pl.BlockSpec((MB, H * HID), lambda m: (m, 0)),
            pl.BlockSpec((MB, C), lambda m: (m, 0)),
        ],
        out_shape=[
            jax.ShapeDtypeStruct((N, H * HID), jnp.float32),
            jax.ShapeDtypeStruct((N, C), jnp.float32),
        ],
    )(msg_p, den_p, res, b, Wc, bc)


def _sc_edge_body(ft_hbm, elr_hbm, src_hbm, dst_hbm, z_hbm,
                  den_out, msg_out, ex_hbm,
                  src_v, dst_v, exc0, exc1, buf_a, buf_b,
                  sem, gs0, gs1, es0, es1, ss0, ss1, acc_sh):
    c = lax.axis_index("c")
    s = lax.axis_index("s")
    wid = c * NS + s
    my_rows = pl.ds(s * ROWS_PER_TILE, ROWS_PER_TILE)
    zvec = jnp.zeros((16,), jnp.float32)

    pltpu.sync_copy(src_hbm.at[wid], src_v)
    pltpu.sync_copy(dst_hbm.at[wid], dst_v)
    pltpu.sync_copy(z_hbm, acc_sh.at[my_rows])
    plsc.subcore_barrier()

    def chunk_a(j, _):
        ca = pltpu.async_copy(elr_hbm.at[src_v.at[j]], buf_a, sem)
        cb = pltpu.async_copy(elr_hbm.at[dst_v.at[j]], buf_b, sem)
        ca.wait()
        cb.wait()

        def edge_a(i, _):
            e = buf_a[i, pl.ds(0, 16)] + buf_b[i, pl.ds(16, 16)]
            e = jnp.where(e > 0, e, SLOPE * e)
            ex = jnp.exp(e)
            lane = lax.iota(jnp.int32, 16)
            ex = jnp.where(lane < 8, ex, 0.0)
            gi = j * CH + i
            vf = jnp.where(gi < EPW, 1.0, 0.0)
            ex = ex * vf
            buf_a[i, pl.ds(0, 16)] = ex
            buf_a[i, pl.ds(16, 16)] = zvec
            exc0[lax.shift_right_logical(i, 3),
                 pl.ds(jnp.bitwise_and(i, 7) * 16, 16)] = ex
            return 0

        lax.fori_loop(0, CH, edge_a, 0, unroll=2)
        pltpu.sync_copy(exc0, ex_hbm.at[wid].at[pl.ds(16 * j, 16)])
        pltpu.sync_copy(buf_a, acc_sh.at[dst_v.at[j]], add=True)
        return 0

    lax.fori_loop(0, NCHUNK, chunk_a, 0)
    plsc.subcore_barrier()
    pltpu.sync_copy(acc_sh.at[my_rows], den_out.at[c].at[my_rows])

    for h in range(H):
        pltpu.sync_copy(z_hbm, acc_sh.at[my_rows])
        plsc.subcore_barrier()

        def g_issue(j, buf, gsem):
            pltpu.async_copy(ft_hbm.at[h].at[src_v.at[j]], buf, gsem)

        def g_wait(j, buf, gsem):
            pltpu.make_async_copy(ft_hbm.at[h].at[src_v.at[j]], buf, gsem).wait()

        def do_chunk(j, jn, buf, gsem, excb):
            pltpu.sync_copy(ex_hbm.at[wid].at[pl.ds(16 * j, 16)], excb)
            g_wait(j, buf, gsem)

            def edge_b(i, _):
                row = excb[lax.shift_right_logical(i, 3),
                           pl.ds(jnp.bitwise_and(i, 7) * 16, 16)]
                a = row[h]
                for s8 in range(H):
                    sl = pl.ds(s8 * 16, 16)
                    buf[i, sl] = buf[i, sl] * a
                return 0

            lax.fori_loop(0, CH, edge_b, 0, unroll=2)
            pltpu.sync_copy(buf, acc_sh.at[dst_v.at[j]], add=True)
            g_issue(jn, buf, gsem)

        g_issue(0, buf_a, gs0)
        g_issue(1, buf_b, gs1)

        @pl.loop(0, NCHUNK, step=2)
        def pair_b(j0):
            j1 = j0 + 1
            do_chunk(j0, jnp.minimum(j0 + 2, NCHUNK - 1), buf_a, gs0, exc0)
            do_chunk(j1, jnp.minimum(j1 + 2, NCHUNK - 1), buf_b, gs1, exc1)
        g_wait(NCHUNK - 1, buf_a, gs0)
        g_wait(NCHUNK - 1, buf_b, gs1)
        plsc.subcore_barrier()
        pltpu.sync_copy(acc_sh.at[my_rows], msg_out.at[c].at[h].at[my_rows])


@functools.cache
def _sc_edge_kernel():
    return pl.kernel(
        _sc_edge_body,
        out_type=[
            jax.ShapeDtypeStruct((NC, NPAD, HID), jnp.float32),
            jax.ShapeDtypeStruct((NC, H, NPAD, HID), jnp.float32),
            jax.ShapeDtypeStruct((NW, EPW_PAD // 8, HID), jnp.float32),
        ],
        mesh=plsc.VectorSubcoreMesh(core_axis_name="c", subcore_axis_name="s"),
        scratch_types=[
            pltpu.VMEM((NCHUNK, CH), jnp.int32),
            pltpu.VMEM((NCHUNK, CH), jnp.int32),
            pltpu.VMEM((CH // 8, HID), jnp.float32),
            pltpu.VMEM((CH // 8, HID), jnp.float32),
            pltpu.VMEM((CH, HID), jnp.float32),
            pltpu.VMEM((CH, HID), jnp.float32),
            pltpu.SemaphoreType.DMA,
            pltpu.SemaphoreType.DMA,
            pltpu.SemaphoreType.DMA,
            pltpu.SemaphoreType.DMA,
            pltpu.SemaphoreType.DMA,
            pltpu.SemaphoreType.DMA,
            pltpu.SemaphoreType.DMA,
            pltpu.VMEM_SHARED((NPAD, HID), jnp.float32),
        ],
    )


def _sc_edge(ft, elr, src, dst, z):
    den, msg, _unused_ex = _sc_edge_kernel()(ft, elr, src, dst, z)
    return den, msg


def _block_diag(a):
    bd = (a[:, :, None] * jnp.eye(H, dtype=a.dtype)[:, None, :]).reshape(H * HID, H)
    return jnp.concatenate([bd, jnp.zeros((H * HID, 8), a.dtype)], axis=1)


def kernel(inputs, target, lamb, edge_index, W1, al1, ar1, b1,
           W2, al2, ar2, b2, Wc, bc):
    A1 = jnp.concatenate([_block_diag(al1), _block_diag(ar1)], axis=1)
    A2 = jnp.concatenate([_block_diag(al2), _block_diag(ar2)], axis=1)

    pad = jnp.zeros((NW, EPW_PAD - EPW), jnp.int32)
    src = jnp.concatenate([edge_index[0].reshape(NW, EPW), pad], axis=1)
    src = src.reshape(NW, NCHUNK, CH)
    dst = jnp.concatenate([edge_index[1].reshape(NW, EPW), pad], axis=1)
    dst = dst.reshape(NW, NCHUNK, CH)

    z = jnp.zeros((ROWS_PER_TILE, HID), jnp.float32)

    ft1, elr1 = _tc1(inputs, W1, A1)
    den1, msg1 = _sc_edge(ft1, elr1, src, dst, z)
    h1, ft2, elr2 = _tc2(msg1, den1, b1.reshape(H, HID), W2, A2)
    den2, msg2 = _sc_edge(ft2, elr2, src, dst, z)
    h2, logits = _tc3(msg2, den2, h1, b2.reshape(H, HID), Wc, bc.reshape(1, C))
    return (h2, logits)


# speedup vs baseline: 1.9952x; 1.9952x over previous
import functools

import jax
import jax.numpy as jnp
from jax import lax
from jax.experimental import pallas as pl
from jax.experimental.pallas import tpu as pltpu
from jax.experimental.pallas import tpu_sc as plsc

N = 10000
E = 160000
IN_DIM = 256
HID = 128
H = 8
C = 40
SLOPE = 0.2

NC = 2
NS = 16
NW = NC * NS
EPW = E // NW
CH = 128
NCHUNK = -(-EPW // CH)
EPW_PAD = NCHUNK * CH
NPAD = 10112
ROWS_PER_TILE = NPAD // NS

MB = 1000
GRID = N // MB


def _tc1_body(x_ref, w_ref, a_ref, ft_ref, elr_ref):
    feat = jnp.dot(x_ref[...], w_ref[...], preferred_element_type=jnp.float32)
    ee = jnp.dot(feat, a_ref[...], preferred_element_type=jnp.float32)
    elr_ref[...] = jnp.concatenate(
        [ee, jnp.zeros((MB, HID - 32), jnp.float32)], axis=1)
    for h in range(H):
        ft_ref[h] = feat[:, h * HID:(h + 1) * HID]


def _tc1(x, W, A):
    return pl.pallas_call(
        _tc1_body,
        grid=(GRID,),
        in_specs=[
            pl.BlockSpec((MB, x.shape[1]), lambda m: (m, 0)),
            pl.BlockSpec((W.shape[0], H * HID), lambda m: (0, 0)),
            pl.BlockSpec((H * HID, 32), lambda m: (0, 0)),
        ],
        out_specs=[
            pl.BlockSpec((H, MB, HID), lambda m: (0, m, 0)),
            pl.BlockSpec((MB, HID), lambda m: (m, 0)),
        ],
        out_shape=[
            jax.ShapeDtypeStruct((H, N, HID), jnp.float32),
            jax.ShapeDtypeStruct((N, HID), jnp.float32),
        ],
    )(x, W, A)


def _combine(msg_p, den_p, b, res):
    den = den_p[0] + den_p[1]
    parts = []
    for h in range(H):
        d = den[:, h:h + 1]
        d = jnp.where(d == 0.0, 1.0, d)
        xh = (msg_p[0, h] + msg_p[1, h]) / d + b[h:h + 1, :]
        if res is not None:
            xh = xh + res[:, h * HID:(h + 1) * HID]
        parts.append(jnp.where(xh > 0, xh, jnp.exp(jnp.minimum(xh, 0.0)) - 1.0))
    return jnp.concatenate(parts, axis=1)


def _tc2_body(mp_ref, dp_ref, b_ref, w_ref, a_ref, h1_ref, ft_ref, elr_ref):
    h1 = _combine(mp_ref[...], dp_ref[...], b_ref[...], None)
    h1_ref[...] = h1
    feat = jnp.dot(h1, w_ref[...], preferred_element_type=jnp.float32)
    ee = jnp.dot(feat, a_ref[...], preferred_element_type=jnp.float32)
    elr_ref[...] = jnp.concatenate(
        [ee, jnp.zeros((MB, HID - 32), jnp.float32)], axis=1)
    for h in range(H):
        ft_ref[h] = feat[:, h * HID:(h + 1) * HID]


def _tc2(msg_p, den_p, b, W, A):
    return pl.pallas_call(
        _tc2_body,
        grid=(GRID,),
        in_specs=[
            pl.BlockSpec((2, H, MB, HID), lambda m: (0, 0, m, 0)),
            pl.BlockSpec((2, MB, HID), lambda m: (0, m, 0)),
            pl.BlockSpec((H, HID), lambda m: (0, 0)),
            pl.BlockSpec((H * HID, H * HID), lambda m: (0, 0)),
            pl.BlockSpec((H * HID, 32), lambda m: (0, 0)),
        ],
        out_specs=[
            pl.BlockSpec((MB, H * HID), lambda m: (m, 0)),
            pl.BlockSpec((H, MB, HID), lambda m: (0, m, 0)),
            pl.BlockSpec((MB, HID), lambda m: (m, 0)),
        ],
        out_shape=[
            jax.ShapeDtypeStruct((N, H * HID), jnp.float32),
            jax.ShapeDtypeStruct((H, N, HID), jnp.float32),
            jax.ShapeDtypeStruct((N, HID), jnp.float32),
        ],
    )(msg_p, den_p, b, W, A)


def _tc3_body(mp_ref, dp_ref, res_ref, b_ref, wc_ref, bc_ref, h_ref, lg_ref):
    hh = _combine(mp_ref[...], dp_ref[...], b_ref[...], res_ref[...])
    h_ref[...] = hh
    lg_ref[...] = jnp.dot(hh, wc_ref[...], preferred_element_type=jnp.float32) + bc_ref[...]


def _tc3(msg_p, den_p, res, b, Wc, bc):
    return pl.pallas_call(
        _tc3_body,
        grid=(GRID,),
        in_specs=[
            pl.BlockSpec((2, H, MB, HID), lambda m: (0, 0, m, 0)),
            pl.BlockSpec((2, MB, HID), lambda m: (0, m, 0)),
            pl.BlockSpec((MB, H * HID), lambda m: (m, 0)),
            pl.BlockSpec((H, HID), lambda m: (0, 0)),
            pl.BlockSpec((H * HID, C), lambda m: (0, 0)),
            pl.BlockSpec((1, C), lambda m: (0, 0)),
        ],
        out_specs=[
            pl.BlockSpec((MB, H * HID), lambda m: (m, 0)),
            pl.BlockSpec((MB, C), lambda m: (m, 0)),
        ],
        out_shape=[
            jax.ShapeDtypeStruct((N, H * HID), jnp.float32),
            jax.ShapeDtypeStruct((N, C), jnp.float32),
        ],
    )(msg_p, den_p, res, b, Wc, bc)


def _sc_edge_body(ft_hbm, elr_hbm, src_hbm, dst_hbm, z_hbm,
                  den_out, msg_out, ex_hbm,
                  src_v, dst_v, exc0, exc1, buf_a, buf_b,
                  sem, gs0, gs1, es0, es1, ss0, ss1, acc_sh):
    c = lax.axis_index("c")
    s = lax.axis_index("s")
    wid = c * NS + s
    my_rows = pl.ds(s * ROWS_PER_TILE, ROWS_PER_TILE)
    zvec = jnp.zeros((16,), jnp.float32)

    pltpu.sync_copy(src_hbm.at[wid], src_v)
    pltpu.sync_copy(dst_hbm.at[wid], dst_v)
    pltpu.sync_copy(z_hbm, acc_sh.at[my_rows])
    plsc.subcore_barrier()

    def chunk_a(j, _):
        ca = pltpu.async_copy(elr_hbm.at[src_v.at[j]], buf_a, sem)
        cb = pltpu.async_copy(elr_hbm.at[dst_v.at[j]], buf_b, sem)
        ca.wait()
        cb.wait()

        def edge_a(i, _):
            e = buf_a[i, pl.ds(0, 16)] + buf_b[i, pl.ds(16, 16)]
            e = jnp.where(e > 0, e, SLOPE * e)
            ex = jnp.exp(e)
            lane = lax.iota(jnp.int32, 16)
            ex = jnp.where(lane < 8, ex, 0.0)
            gi = j * CH + i
            vf = jnp.where(gi < EPW, 1.0, 0.0)
            ex = ex * vf
            buf_a[i, pl.ds(0, 16)] = ex
            buf_a[i, pl.ds(16, 16)] = zvec
            exc0[lax.shift_right_logical(i, 3),
                 pl.ds(jnp.bitwise_and(i, 7) * 16, 16)] = ex
            return 0

        lax.fori_loop(0, CH, edge_a, 0, unroll=2)
        pltpu.sync_copy(exc0, ex_hbm.at[wid].at[pl.ds(16 * j, 16)])
        pltpu.sync_copy(buf_a, acc_sh.at[dst_v.at[j]], add=True)
        return 0

    lax.fori_loop(0, NCHUNK, chunk_a, 0)
    plsc.subcore_barrier()
    pltpu.sync_copy(acc_sh.at[my_rows], den_out.at[c].at[my_rows])

    RING = 8
    for h in range(H):
        pltpu.sync_copy(z_hbm, acc_sh.at[my_rows])
        plsc.subcore_barrier()

        def issue(j, buf, excb):
            g = pltpu.async_copy(ft_hbm.at[h].at[src_v.at[j]], buf, gs0)
            e = pltpu.async_copy(ex_hbm.at[wid].at[pl.ds(16 * j, 16)],
                                 excb, es0)
            return (g, e)

        @pl.loop(0, NCHUNK, step=RING)
        def blk(j0):
            bufs = (buf_a, buf_b)
            excs = (exc0, exc1)
            d = {}
            d[0] = issue(j0, buf_a, exc0)
            d[1] = issue(j0 + 1, buf_b, exc1)
            for m in range(RING):
                buf = bufs[m % 2]
                excb = excs[m % 2]
                j = j0 + m
                g, e = d[m]
                g.wait()
                e.wait()

                def edge_b(i, _):
                    row = excb[lax.shift_right_logical(i, 3),
                               pl.ds(jnp.bitwise_and(i, 7) * 16, 16)]
                    a = row[h]
                    for s8 in range(H):
                        sl = pl.ds(s8 * 16, 16)
                        buf[i, sl] = buf[i, sl] * a
                    return 0

                lax.fori_loop(0, CH, edge_b, 0, unroll=2)
                pltpu.sync_copy(buf, acc_sh.at[dst_v.at[j]], add=True)
                if m + 2 < RING:
                    d[m + 2] = issue(j + 2, buf, excb)

        plsc.subcore_barrier()
        pltpu.sync_copy(acc_sh.at[my_rows], msg_out.at[c].at[h].at[my_rows])


@functools.cache
def _sc_edge_kernel():
    return pl.kernel(
        _sc_edge_body,
        out_type=[
            jax.ShapeDtypeStruct((NC, NPAD, HID), jnp.float32),
            jax.ShapeDtypeStruct((NC, H, NPAD, HID), jnp.float32),
            jax.ShapeDtypeStruct((NW, EPW_PAD // 8, HID), jnp.float32),
        ],
        mesh=plsc.VectorSubcoreMesh(core_axis_name="c", subcore_axis_name="s"),
        scratch_types=[
            pltpu.VMEM((NCHUNK, CH), jnp.int32),
            pltpu.VMEM((NCHUNK, CH), jnp.int32),
            pltpu.VMEM((CH // 8, HID), jnp.float32),
            pltpu.VMEM((CH // 8, HID), jnp.float32),
            pltpu.VMEM((CH, HID), jnp.float32),
            pltpu.VMEM((CH, HID), jnp.float32),
            pltpu.SemaphoreType.DMA,
            pltpu.SemaphoreType.DMA,
            pltpu.SemaphoreType.DMA,
            pltpu.SemaphoreType.DMA,
            pltpu.SemaphoreType.DMA,
            pltpu.SemaphoreType.DMA,
            pltpu.SemaphoreType.DMA,
            pltpu.VMEM_SHARED((NPAD, HID), jnp.float32),
        ],
    )


def _sc_edge(ft, elr, src, dst, z):
    den, msg, _unused_ex = _sc_edge_kernel()(ft, elr, src, dst, z)
    return den, msg


def _block_diag(a):
    bd = (a[:, :, None] * jnp.eye(H, dtype=a.dtype)[:, None, :]).reshape(H * HID, H)
    return jnp.concatenate([bd, jnp.zeros((H * HID, 8), a.dtype)], axis=1)


def kernel(inputs, target, lamb, edge_index, W1, al1, ar1, b1,
           W2, al2, ar2, b2, Wc, bc):
    A1 = jnp.concatenate([_block_diag(al1), _block_diag(ar1)], axis=1)
    A2 = jnp.concatenate([_block_diag(al2), _block_diag(ar2)], axis=1)

    pad = jnp.zeros((NW, EPW_PAD - EPW), jnp.int32)
    src = jnp.concatenate([edge_index[0].reshape(NW, EPW), pad], axis=1)
    src = src.reshape(NW, NCHUNK, CH)
    dst = jnp.concatenate([edge_index[1].reshape(NW, EPW), pad], axis=1)
    dst = dst.reshape(NW, NCHUNK, CH)

    z = jnp.zeros((ROWS_PER_TILE, HID), jnp.float32)

    ft1, elr1 = _tc1(inputs, W1, A1)
    den1, msg1 = _sc_edge(ft1, elr1, src, dst, z)
    h1, ft2, elr2 = _tc2(msg1, den1, b1.reshape(H, HID), W2, A2)
    den2, msg2 = _sc_edge(ft2, elr2, src, dst, z)
    h2, logits = _tc3(msg2, den2, h1, b2.reshape(H, HID), Wc, bc.reshape(1, C))
    return (h2, logits)
